# baseline (speedup 1.0000x reference)
import functools

import jax
import jax.numpy as jnp
from jax import lax
from jax.experimental import pallas as pl
from jax.experimental.pallas import tpu as pltpu
from jax.experimental.pallas import tpu_sc as plsc

NUM_CLASSES = 1000
N_CTX = 16
CTX_DIM = 768
SEQ_LEN = 77
BATCH = 1024

N_SUF = SEQ_LEN - 1 - N_CTX

_info = plsc.get_sparse_core_info()
NC, NS, NL = _info.num_cores, _info.num_subcores, _info.num_lanes
NW = NC * NS
BPW = BATCH // NW

_NRING = 4

_mesh = plsc.VectorSubcoreMesh(core_axis_name="c", subcore_axis_name="s")


@functools.partial(
    pl.kernel,
    out_type=jax.ShapeDtypeStruct((BATCH, N_CTX, CTX_DIM), jnp.float32),
    mesh=_mesh,
    compiler_params=pltpu.CompilerParams(
        use_tc_tiling_on_sc=True, needs_layout_passes=False),
    scratch_types=(
        [pltpu.VMEM((BPW,), jnp.int32)]
        + [pltpu.VMEM((N_CTX, CTX_DIM), jnp.float32)] * _NRING
        + [pltpu.SemaphoreType.DMA] * (2 * _NRING)
    ),
)
def _gather_ctx_sc(label_hbm, ctx_hbm, gctx_hbm, idx_v, *bufs_sems):
    bufs = bufs_sems[:_NRING]
    gsems = bufs_sems[_NRING:2 * _NRING]
    osems = bufs_sems[2 * _NRING:]

    wid = lax.axis_index("s") * NC + lax.axis_index("c")
    base = wid * BPW

    pltpu.sync_copy(label_hbm.at[pl.ds(base, BPW)], idx_v)

    lanes = lax.iota(jnp.int32, NL)
    chunks = [idx_v[pl.ds(k * NL, NL)] for k in range(BPW // NL)]

    def label_of(e):
        v = jnp.where(lanes == (e % NL), chunks[e // NL], 0)
        return lax.reduce_max(v, (0,))

    labels = [label_of(e) for e in range(BPW)]

    def cin(e):
        s = e % _NRING
        return pltpu.make_async_copy(
            ctx_hbm.at[labels[e]], bufs[s], gsems[s])

    def cout(e):
        s = e % _NRING
        return pltpu.make_async_copy(
            bufs[s], gctx_hbm.at[base + e], osems[s])

    for e in range(_NRING):
        cin(e).start()

    for e in range(BPW):
        cin(e).wait()
        cout(e).start()
        if e + _NRING < BPW:
            cout(e).wait()
            cin(e + _NRING).start()
        else:
            cout(e).wait()


def _concat_tc(lbl_ref, pre_ref, gctx_ref, suf_ref, out_ref):
    del lbl_ref
    out_ref[:, 0:1, :] = pre_ref[...]
    out_ref[:, 1:1 + N_CTX, :] = gctx_ref[...]
    out_ref[:, 1 + N_CTX:SEQ_LEN, :] = suf_ref[...]


_assemble_tc = pl.pallas_call(
    _concat_tc,
    out_shape=jax.ShapeDtypeStruct((BATCH, SEQ_LEN, CTX_DIM), jnp.float32),
    grid_spec=pltpu.PrefetchScalarGridSpec(
        num_scalar_prefetch=1,
        grid=(BATCH,),
        in_specs=[
            pl.BlockSpec((1, 1, CTX_DIM), lambda b, lbl: (0, 0, 0)),
            pl.BlockSpec((1, N_CTX, CTX_DIM), lambda b, lbl: (b, 0, 0)),
            pl.BlockSpec((1, N_SUF, CTX_DIM), lambda b, lbl: (lbl[b], 0, 0)),
        ],
        out_specs=pl.BlockSpec(
            (1, SEQ_LEN, CTX_DIM), lambda b, lbl: (b, 0, 0)),
    ),
)


@jax.jit
def kernel(label, cls_ctx, token_prefix, token_suffix):
    lab = label.astype(jnp.int32)
    gctx = _gather_ctx_sc(lab, cls_ctx)
    return _assemble_tc(lab, token_prefix, gctx, token_suffix)


# speedup vs baseline: 1.8884x; 1.8884x over previous
import functools

import jax
import jax.numpy as jnp
from jax import lax
from jax.experimental import pallas as pl
from jax.experimental.pallas import tpu as pltpu
from jax.experimental.pallas import tpu_sc as plsc

NUM_CLASSES = 1000
N_CTX = 16
CTX_DIM = 768
SEQ_LEN = 77
BATCH = 1024

N_SUF = SEQ_LEN - 1 - N_CTX

_info = plsc.get_sparse_core_info()
NC, NS, NL = _info.num_cores, _info.num_subcores, _info.num_lanes
NW = NC * NS
BPW = BATCH // NW

_NRING = 4

_mesh = plsc.VectorSubcoreMesh(core_axis_name="c", subcore_axis_name="s")


@functools.partial(
    pl.kernel,
    out_type=jax.ShapeDtypeStruct((BATCH, N_CTX, CTX_DIM), jnp.float32),
    mesh=_mesh,
    compiler_params=pltpu.CompilerParams(
        use_tc_tiling_on_sc=True, needs_layout_passes=False),
    scratch_types=(
        [pltpu.VMEM((BPW,), jnp.int32)]
        + [pltpu.VMEM((N_CTX, CTX_DIM), jnp.float32)] * _NRING
        + [pltpu.SemaphoreType.DMA] * (2 * _NRING)
    ),
)
def _gather_ctx_sc(label_hbm, ctx_hbm, gctx_hbm, idx_v, *bufs_sems):
    bufs = bufs_sems[:_NRING]
    gsems = bufs_sems[_NRING:2 * _NRING]
    osems = bufs_sems[2 * _NRING:]

    wid = lax.axis_index("s") * NC + lax.axis_index("c")
    base = wid * BPW

    pltpu.sync_copy(label_hbm.at[pl.ds(base, BPW)], idx_v)

    lanes = lax.iota(jnp.int32, NL)
    chunks = [idx_v[pl.ds(k * NL, NL)] for k in range(BPW // NL)]

    def label_of(e):
        v = jnp.where(lanes == (e % NL), chunks[e // NL], 0)
        return lax.reduce_max(v, (0,))

    labels = [label_of(e) for e in range(BPW)]

    def cin(e):
        s = e % _NRING
        return pltpu.make_async_copy(
            ctx_hbm.at[labels[e]], bufs[s], gsems[s])

    def cout(e):
        s = e % _NRING
        return pltpu.make_async_copy(
            bufs[s], gctx_hbm.at[base + e], osems[s])

    for e in range(_NRING):
        cin(e).start()

    for e in range(BPW):
        cin(e).wait()
        cout(e).start()
        if e + _NRING < BPW:
            cout(e).wait()
            cin(e + _NRING).start()
        else:
            cout(e).wait()


B_TC = 8


def _concat_tc(lbl_ref, pre_ref, gctx_ref, *suf_and_out):
    del lbl_ref
    suf_refs = suf_and_out[:B_TC]
    out_ref = suf_and_out[B_TC]
    out_ref[:, 0:1, :] = jnp.broadcast_to(pre_ref[...], (B_TC, 1, CTX_DIM))
    out_ref[:, 1:1 + N_CTX, :] = gctx_ref[...]
    for j in range(B_TC):
        out_ref[j:j + 1, 1 + N_CTX:SEQ_LEN, :] = suf_refs[j][...]


_assemble_tc = pl.pallas_call(
    _concat_tc,
    out_shape=jax.ShapeDtypeStruct((BATCH, SEQ_LEN, CTX_DIM), jnp.float32),
    grid_spec=pltpu.PrefetchScalarGridSpec(
        num_scalar_prefetch=1,
        grid=(BATCH // B_TC,),
        in_specs=[
            pl.BlockSpec((1, 1, CTX_DIM), lambda i, lbl: (0, 0, 0)),
            pl.BlockSpec((B_TC, N_CTX, CTX_DIM), lambda i, lbl: (i, 0, 0)),
        ] + [
            pl.BlockSpec(
                (1, N_SUF, CTX_DIM),
                functools.partial(
                    lambda j, i, lbl: (lbl[i * B_TC + j], 0, 0), j))
            for j in range(B_TC)
        ],
        out_specs=pl.BlockSpec(
            (B_TC, SEQ_LEN, CTX_DIM), lambda i, lbl: (i, 0, 0)),
    ),
)


@jax.jit
def kernel(label, cls_ctx, token_prefix, token_suffix):
    lab = label.astype(jnp.int32)
    gctx = _gather_ctx_sc(lab, cls_ctx)
    return _assemble_tc(
        lab, token_prefix, gctx, *([token_suffix] * B_TC))
